# baseline (speedup 1.0000x reference)
import functools

import jax
import jax.numpy as jnp
from jax import lax
from jax.experimental import pallas as pl
from jax.experimental.pallas import tpu as pltpu
from jax.experimental.pallas import tpu_sc as plsc

BINS = 128
NF = 128
NB = 16
SEQ = 8192
CHUNK = 256
ROWS_PER_TILE = NB * SEQ // 32
NCHUNK = ROWS_PER_TILE // CHUNK
LANES = 16


def _sc_hist(inp2):
    mesh = plsc.VectorSubcoreMesh(core_axis_name="c", subcore_axis_name="s")

    @functools.partial(
        pl.kernel,
        out_type=jax.ShapeDtypeStruct((2, 16, NF * BINS), jnp.float32),
        mesh=mesh,
        compiler_params=pltpu.CompilerParams(needs_layout_passes=False),
        scratch_types=[
            pltpu.VMEM((CHUNK, NF), jnp.float32),
            pltpu.VMEM((CHUNK, NF), jnp.float32),
            pltpu.VMEM((NF * BINS,), jnp.float32),
            pltpu.SemaphoreType.DMA,
            pltpu.SemaphoreType.DMA,
        ],
    )
    def k(inp_hbm, out_hbm, buf0, buf1, hist, sem0, sem1):
        c = lax.axis_index("c")
        s = lax.axis_index("s")
        base = s * SEQ + c * ROWS_PER_TILE

        zeros16 = jnp.zeros((LANES,), jnp.float32)
        ones16 = jnp.ones((LANES,), jnp.float32)
        lane = lax.iota(jnp.int32, LANES)
        bases = [lane + v * BINS * LANES for v in range(NF // LANES)]

        def zbody(i, carry):
            hist[pl.ds(i * LANES, LANES)] = zeros16
            return carry

        lax.fori_loop(0, NF * BINS // LANES, zbody, 0)

        bufs = [buf0, buf1]
        sems = [sem0, sem1]
        copies = [None] * NCHUNK
        copies[0] = pltpu.async_copy(
            inp_hbm.at[pl.ds(base, CHUNK)], buf0, sem0)
        for ch in range(NCHUNK):
            if ch + 1 < NCHUNK:
                copies[ch + 1] = pltpu.async_copy(
                    inp_hbm.at[pl.ds(base + (ch + 1) * CHUNK, CHUNK)],
                    bufs[(ch + 1) % 2], sems[(ch + 1) % 2])
            copies[ch].wait()
            buf = bufs[ch % 2]

            @plsc.parallel_loop(0, CHUNK, unroll=2)
            def rbody(r, buf=buf):
                for v in range(NF // LANES):
                    x = buf[r, pl.ds(v * LANES, LANES)]
                    idx = (x * float(BINS)).astype(jnp.int32)
                    plsc.addupdate_scatter(
                        hist, [(idx << 4) + bases[v]], ones16)

        pltpu.sync_copy(hist, out_hbm.at[c, s])

    return k(inp2)


def _tc_finish(hp, weight):
    def body(h_ref, w_ref, o_ref):
        for v in range(NF // LANES):
            sm = h_ref[0, 0, v] + h_ref[1, 0, v]
            o_ref[0, :, v * LANES:(v + 1) * LANES] = (
                sm * w_ref[:, v * LANES:(v + 1) * LANES])

    return pl.pallas_call(
        body,
        grid=(NB,),
        in_specs=[
            pl.BlockSpec((2, 1, NF // LANES, BINS, LANES),
                         lambda b: (0, b, 0, 0, 0)),
            pl.BlockSpec((BINS, NF), lambda b: (0, 0)),
        ],
        out_specs=pl.BlockSpec((1, BINS, NF), lambda b: (b, 0, 0)),
        out_shape=jax.ShapeDtypeStruct((NB, BINS, NF), jnp.float32),
    )(hp, weight)


def kernel(input, weight):
    bs, seq, fs = input.shape
    assert (bs, seq, fs) == (NB, SEQ, NF) and weight.shape == (BINS, NF)
    inp2 = input.reshape(bs * seq, fs)
    hist = _sc_hist(inp2)
    hp = hist.reshape(2, NB, NF // LANES, BINS, LANES)
    return _tc_finish(hp, weight)


# speedup vs baseline: 1.1250x; 1.1250x over previous
import functools

import jax
import jax.numpy as jnp
from jax import lax
from jax.experimental import pallas as pl
from jax.experimental.pallas import tpu as pltpu
from jax.experimental.pallas import tpu_sc as plsc

BINS = 128
NF = 128
NB = 16
SEQ = 8192
CHUNK = 256
ROWS_PER_TILE = NB * SEQ // 32
NCHUNK = ROWS_PER_TILE // CHUNK
LANES = 16


def _sc_hist(inp2):
    mesh = plsc.VectorSubcoreMesh(core_axis_name="c", subcore_axis_name="s")

    @functools.partial(
        pl.kernel,
        out_type=jax.ShapeDtypeStruct((2, 16, NF * BINS), jnp.float32),
        mesh=mesh,
        compiler_params=pltpu.CompilerParams(needs_layout_passes=False),
        scratch_types=[
            pltpu.VMEM((CHUNK, NF), jnp.float32),
            pltpu.VMEM((CHUNK, NF), jnp.float32),
            pltpu.VMEM((NF * BINS,), jnp.float32),
            pltpu.SemaphoreType.DMA,
            pltpu.SemaphoreType.DMA,
        ],
    )
    def k(inp_hbm, out_hbm, buf0, buf1, hist, sem0, sem1):
        c = lax.axis_index("c")
        s = lax.axis_index("s")
        base = s * SEQ + c * ROWS_PER_TILE

        zeros16 = jnp.zeros((LANES,), jnp.float32)
        ones16 = jnp.ones((LANES,), jnp.float32)
        lane = lax.iota(jnp.int32, LANES)
        bases = [lane + v * LANES for v in range(NF // LANES)]

        def zbody(i, carry):
            hist[pl.ds(i * LANES, LANES)] = zeros16
            return carry

        lax.fori_loop(0, NF * BINS // LANES, zbody, 0)

        bufs = [buf0, buf1]
        sems = [sem0, sem1]
        copies = [None] * NCHUNK
        copies[0] = pltpu.async_copy(
            inp_hbm.at[pl.ds(base, CHUNK)], buf0, sem0)
        for ch in range(NCHUNK):
            if ch + 1 < NCHUNK:
                copies[ch + 1] = pltpu.async_copy(
                    inp_hbm.at[pl.ds(base + (ch + 1) * CHUNK, CHUNK)],
                    bufs[(ch + 1) % 2], sems[(ch + 1) % 2])
            copies[ch].wait()
            buf = bufs[ch % 2]

            @plsc.parallel_loop(0, CHUNK, unroll=2)
            def rbody(r, buf=buf):
                for v in range(NF // LANES):
                    x = buf[r, pl.ds(v * LANES, LANES)]
                    idx = (x * float(BINS)).astype(jnp.int32)
                    plsc.addupdate_scatter(
                        hist, [(idx << 7) + bases[v]], ones16)

        pltpu.sync_copy(hist, out_hbm.at[c, s])

    return k(inp2)


def _tc_finish(hp, weight):
    def body(h_ref, w_ref, o_ref):
        o_ref[0] = (h_ref[0, 0] + h_ref[1, 0]) * w_ref[...]

    return pl.pallas_call(
        body,
        grid=(NB,),
        in_specs=[
            pl.BlockSpec((2, 1, BINS, NF), lambda b: (0, b, 0, 0)),
            pl.BlockSpec((BINS, NF), lambda b: (0, 0)),
        ],
        out_specs=pl.BlockSpec((1, BINS, NF), lambda b: (b, 0, 0)),
        out_shape=jax.ShapeDtypeStruct((NB, BINS, NF), jnp.float32),
    )(hp, weight)


def kernel(input, weight):
    bs, seq, fs = input.shape
    assert (bs, seq, fs) == (NB, SEQ, NF) and weight.shape == (BINS, NF)
    inp2 = input.reshape(bs * seq, fs)
    hist = _sc_hist(inp2)
    hp = hist.reshape(2, NB, BINS, NF)
    return _tc_finish(hp, weight)


# speedup vs baseline: 1.3055x; 1.1604x over previous
import functools

import jax
import jax.numpy as jnp
from jax import lax
from jax.experimental import pallas as pl
from jax.experimental.pallas import tpu as pltpu
from jax.experimental.pallas import tpu_sc as plsc

BINS = 128
NF = 128
NB = 16
SEQ = 8192
CHUNK = 256
ROWS_PER_TILE = NB * SEQ // 32
NCHUNK = ROWS_PER_TILE // CHUNK
LANES = 16
HWORDS = BINS * NF // 2


def _sc_hist(inp2, weight):
    mesh = plsc.VectorSubcoreMesh(core_axis_name="c", subcore_axis_name="s")

    @functools.partial(
        pl.kernel,
        out_type=jax.ShapeDtypeStruct((NB, BINS, NF), jnp.float32),
        mesh=mesh,
        compiler_params=pltpu.CompilerParams(needs_layout_passes=False),
        scratch_types=[
            pltpu.VMEM((CHUNK, NF), jnp.float32),
            pltpu.VMEM((CHUNK, NF), jnp.float32),
            pltpu.VMEM((BINS, NF), jnp.float32),
            pltpu.VMEM_SHARED((16, BINS // 2, NF), jnp.float32),
            pltpu.SemaphoreType.DMA,
            pltpu.SemaphoreType.DMA,
        ],
    )
    def k(inp_hbm, w_hbm, out_hbm, buf0, buf1, hist, shared, sem0, sem1):
        c = lax.axis_index("c")
        s = lax.axis_index("s")
        b = c * 8 + (s // 2)
        b_local = s // 2
        half = s % 2
        base = b * SEQ + half * ROWS_PER_TILE

        zeros16 = jnp.zeros((LANES,), jnp.float32)
        ones16 = jnp.ones((LANES,), jnp.float32)
        lane = lax.iota(jnp.int32, LANES)
        bases = [lane + v * LANES for v in range(NF // LANES)]

        bufs = [buf0, buf1]
        sems = [sem0, sem1]
        copies = [None] * NCHUNK
        copies[0] = pltpu.async_copy(
            inp_hbm.at[pl.ds(base, CHUNK)], buf0, sem0)

        @plsc.parallel_loop(0, BINS, unroll=4)
        def zbody(i):
            for v in range(NF // LANES):
                hist[i, pl.ds(v * LANES, LANES)] = zeros16

        for ch in range(NCHUNK):
            if ch + 1 < NCHUNK:
                copies[ch + 1] = pltpu.async_copy(
                    inp_hbm.at[pl.ds(base + (ch + 1) * CHUNK, CHUNK)],
                    bufs[(ch + 1) % 2], sems[(ch + 1) % 2])
            copies[ch].wait()
            buf = bufs[ch % 2]

            @plsc.parallel_loop(0, CHUNK, unroll=2)
            def rbody(r, buf=buf):
                for v in range(NF // LANES):
                    x = buf[r, pl.ds(v * LANES, LANES)]
                    idx = (x * float(BINS)).astype(jnp.int32)
                    plsc.addupdate_scatter(hist, [idx, bases[v]], ones16)

        nhalf = 1 - half
        hb = BINS // 2
        pltpu.sync_copy(hist.at[pl.ds(nhalf * hb, hb)], shared.at[s])
        wbuf = buf0.at[pl.ds(0, hb)]
        pltpu.sync_copy(w_hbm.at[pl.ds(half * hb, hb)], wbuf)
        plsc.subcore_barrier()
        pbuf = buf1.at[pl.ds(0, hb)]
        pltpu.sync_copy(shared.at[s ^ 1], pbuf)

        @plsc.parallel_loop(0, hb, unroll=2)
        def mbody(r):
            for v in range(NF // LANES):
                off = pl.ds(v * LANES, LANES)
                hist[half * hb + r, off] = (
                    (hist[half * hb + r, off] + buf1[r, off])
                    * buf0[r, off])

        pltpu.sync_copy(hist.at[pl.ds(half * hb, hb)],
                        out_hbm.at[b, pl.ds(half * hb, hb)])

    return k(inp2, weight)


def kernel(input, weight):
    bs, seq, fs = input.shape
    assert (bs, seq, fs) == (NB, SEQ, NF) and weight.shape == (BINS, NF)
    inp2 = input.reshape(bs * seq, fs)
    return _sc_hist(inp2, weight)


# speedup vs baseline: 1.3062x; 1.0005x over previous
import functools

import jax
import jax.numpy as jnp
from jax import lax
from jax.experimental import pallas as pl
from jax.experimental.pallas import tpu as pltpu
from jax.experimental.pallas import tpu_sc as plsc

BINS = 128
NF = 128
NB = 16
SEQ = 8192
CHUNK = 256
ROWS_PER_TILE = NB * SEQ // 32
NCHUNK = ROWS_PER_TILE // CHUNK
LANES = 16
HWORDS = BINS * NF // 2


def _sc_hist(inp, weight):
    mesh = plsc.VectorSubcoreMesh(core_axis_name="c", subcore_axis_name="s")

    @functools.partial(
        pl.kernel,
        out_type=jax.ShapeDtypeStruct((NB, BINS, NF), jnp.float32),
        mesh=mesh,
        compiler_params=pltpu.CompilerParams(needs_layout_passes=False),
        scratch_types=[
            pltpu.VMEM((CHUNK, NF), jnp.float32),
            pltpu.VMEM((CHUNK, NF), jnp.float32),
            pltpu.VMEM((BINS, NF), jnp.float32),
            pltpu.VMEM_SHARED((16, BINS // 2, NF), jnp.float32),
            pltpu.SemaphoreType.DMA,
            pltpu.SemaphoreType.DMA,
        ],
    )
    def k(inp_hbm, w_hbm, out_hbm, buf0, buf1, hist, shared, sem0, sem1):
        c = lax.axis_index("c")
        s = lax.axis_index("s")
        b = c * 8 + (s // 2)
        half = s % 2
        base = half * ROWS_PER_TILE

        zeros16 = jnp.zeros((LANES,), jnp.float32)
        ones16 = jnp.ones((LANES,), jnp.float32)
        lane = lax.iota(jnp.int32, LANES)
        bases = [lane + v * LANES for v in range(NF // LANES)]

        bufs = [buf0, buf1]
        sems = [sem0, sem1]
        copies = [None] * NCHUNK
        copies[0] = pltpu.async_copy(
            inp_hbm.at[b, pl.ds(base, CHUNK)], buf0, sem0)

        @plsc.parallel_loop(0, BINS, unroll=4)
        def zbody(i):
            for v in range(NF // LANES):
                hist[i, pl.ds(v * LANES, LANES)] = zeros16

        for ch in range(NCHUNK):
            if ch + 1 < NCHUNK:
                copies[ch + 1] = pltpu.async_copy(
                    inp_hbm.at[b, pl.ds(base + (ch + 1) * CHUNK, CHUNK)],
                    bufs[(ch + 1) % 2], sems[(ch + 1) % 2])
            copies[ch].wait()
            buf = bufs[ch % 2]

            @plsc.parallel_loop(0, CHUNK, unroll=2)
            def rbody(r, buf=buf):
                for v in range(NF // LANES):
                    x = buf[r, pl.ds(v * LANES, LANES)]
                    idx = (x * float(BINS)).astype(jnp.int32)
                    plsc.addupdate_scatter(hist, [idx, bases[v]], ones16)

        nhalf = 1 - half
        hb = BINS // 2
        pltpu.sync_copy(hist.at[pl.ds(nhalf * hb, hb)], shared.at[s])
        wbuf = buf0.at[pl.ds(0, hb)]
        pltpu.sync_copy(w_hbm.at[pl.ds(half * hb, hb)], wbuf)
        plsc.subcore_barrier()
        pbuf = buf1.at[pl.ds(0, hb)]
        pltpu.sync_copy(shared.at[s ^ 1], pbuf)

        @plsc.parallel_loop(0, hb, unroll=2)
        def mbody(r):
            for v in range(NF // LANES):
                off = pl.ds(v * LANES, LANES)
                hist[half * hb + r, off] = (
                    (hist[half * hb + r, off] + buf1[r, off])
                    * buf0[r, off])

        pltpu.sync_copy(hist.at[pl.ds(half * hb, hb)],
                        out_hbm.at[b, pl.ds(half * hb, hb)])

    return k(inp, weight)


def kernel(input, weight):
    bs, seq, fs = input.shape
    assert (bs, seq, fs) == (NB, SEQ, NF) and weight.shape == (BINS, NF)
    return _sc_hist(input, weight)


# speedup vs baseline: 1.3064x; 1.0001x over previous
import functools

import jax
import jax.numpy as jnp
from jax import lax
from jax.experimental import pallas as pl
from jax.experimental.pallas import tpu as pltpu
from jax.experimental.pallas import tpu_sc as plsc

BINS = 128
NF = 128
NB = 16
SEQ = 8192
CHUNK = 256
ROWS_PER_TILE = NB * SEQ // 32
NCHUNK = ROWS_PER_TILE // CHUNK
LANES = 16


def _sc_hist(inp2, weight):
    mesh = plsc.VectorSubcoreMesh(core_axis_name="c", subcore_axis_name="s")

    @functools.partial(
        pl.kernel,
        out_type=jax.ShapeDtypeStruct((NB, BINS, NF), jnp.float32),
        mesh=mesh,
        compiler_params=pltpu.CompilerParams(needs_layout_passes=False),
        scratch_types=[
            pltpu.VMEM((CHUNK, NF), jnp.float32),
            pltpu.VMEM((CHUNK, NF), jnp.float32),
            pltpu.VMEM((BINS, NF), jnp.float32),
            pltpu.VMEM_SHARED((16, BINS // 2, NF), jnp.float32),
            pltpu.SemaphoreType.DMA,
            pltpu.SemaphoreType.DMA,
        ],
    )
    def k(inp_hbm, w_hbm, out_hbm, buf0, buf1, hist, shared, sem0, sem1):
        c = lax.axis_index("c")
        s = lax.axis_index("s")
        b = c * 8 + (s // 2)
        half = s % 2
        base = b * SEQ + half * ROWS_PER_TILE

        zeros16 = jnp.zeros((LANES,), jnp.float32)
        ones16 = jnp.ones((LANES,), jnp.float32)
        lane = lax.iota(jnp.int32, LANES)
        bases = [lane + v * LANES for v in range(NF // LANES)]

        bufs = [buf0, buf1]
        sems = [sem0, sem1]
        copies = [None] * NCHUNK
        copies[0] = pltpu.async_copy(
            inp_hbm.at[pl.ds(base, CHUNK)], buf0, sem0)

        @plsc.parallel_loop(0, BINS, unroll=4)
        def zbody(i):
            for v in range(NF // LANES):
                hist[i, pl.ds(v * LANES, LANES)] = zeros16

        for ch in range(NCHUNK):
            if ch + 1 < NCHUNK:
                copies[ch + 1] = pltpu.async_copy(
                    inp_hbm.at[pl.ds(base + (ch + 1) * CHUNK, CHUNK)],
                    bufs[(ch + 1) % 2], sems[(ch + 1) % 2])
            copies[ch].wait()
            buf = bufs[ch % 2]

            @plsc.parallel_loop(0, CHUNK, unroll=2)
            def rbody(r, buf=buf):
                for v in range(NF // LANES):
                    x = buf[r, pl.ds(v * LANES, LANES)]
                    idx = (x * float(BINS)).astype(jnp.int32)
                    plsc.addupdate_scatter(hist, [idx, bases[v]], ones16)

        nhalf = 1 - half
        hb = BINS // 2
        pltpu.sync_copy(hist.at[pl.ds(nhalf * hb, hb)], shared.at[s])
        wbuf = buf0.at[pl.ds(0, hb)]
        pltpu.sync_copy(w_hbm.at[pl.ds(half * hb, hb)], wbuf)
        plsc.subcore_barrier()
        pbuf = buf1.at[pl.ds(0, hb)]
        pltpu.sync_copy(shared.at[s ^ 1], pbuf)

        @plsc.parallel_loop(0, hb, unroll=2)
        def mbody(r):
            for v in range(NF // LANES):
                off = pl.ds(v * LANES, LANES)
                hist[half * hb + r, off] = (
                    (hist[half * hb + r, off] + buf1[r, off])
                    * buf0[r, off])

        pltpu.sync_copy(hist.at[pl.ds(half * hb, hb)],
                        out_hbm.at[b, pl.ds(half * hb, hb)])

    return k(inp2, weight)


def kernel(input, weight):
    bs, seq, fs = input.shape
    assert (bs, seq, fs) == (NB, SEQ, NF) and weight.shape == (BINS, NF)
    inp2 = input.reshape(bs * seq, fs)
    return _sc_hist(inp2, weight)
